# Initial kernel scaffold; baseline (speedup 1.0000x reference)
import jax
import jax.numpy as jnp
from jax.experimental import pallas as pl


def kernel(x, edge_index, W1, b1, W2, b2):
    raise NotImplementedError("write your pallas kernel here")


# speedup vs baseline: 11.6132x; 11.6132x over previous
import functools

import jax
import jax.numpy as jnp
from jax import lax
from jax.experimental import pallas as pl
from jax.experimental.pallas import tpu as pltpu
from jax.experimental.pallas import tpu_sc as plsc

NC = 2
NS = 16
NW = NC * NS
K = 128
DEG_W = 32


def _fill_rows(ref, rows, width, value):
    def body(j, carry):
        for l in range(width // 16):
            ref[j, pl.ds(l * 16, 16)] = jnp.full((16,), value, jnp.float32)
        return carry
    lax.fori_loop(0, rows, body, 0)


@functools.lru_cache(maxsize=None)
def _make_deg(N, E_pad):
    EPT = E_pad // NW
    steps = EPT // K
    NR = ((N + 1 + NS * K - 1) // (NS * K)) * (NS * K)
    RZ = NR // NS
    RO_STRIDE = 8 * (N // (8 * NS))
    RO_LEN = N - (NS - 1) * RO_STRIDE
    mesh = plsc.VectorSubcoreMesh(core_axis_name="c", subcore_axis_name="s", num_cores=NC, num_subcores=NS)

    @functools.partial(
        pl.kernel,
        out_type=jax.ShapeDtypeStruct((NC, N, DEG_W), jnp.float32),
        mesh=mesh,
        scratch_types=[
            pltpu.VMEM((K,), jnp.int32),
            pltpu.VMEM((K, DEG_W), jnp.float32),
            pltpu.VMEM_SHARED((NR, DEG_W), jnp.float32),
        ],
    )
    def deg_kernel(dstp_hbm, out_hbm, didx, buf, acc):
        c = lax.axis_index("c")
        s = lax.axis_index("s")
        tid = c * NS + s
        _fill_rows(buf, K, DEG_W, 0.0)
        for i in range(RZ // K):
            pltpu.sync_copy(buf, acc.at[pl.ds(s * RZ + i * K, K)])
        _fill_rows(buf, K, DEG_W, 1.0)
        plsc.subcore_barrier()

        def step(i, carry):
            e0 = tid * EPT + i * K
            pltpu.sync_copy(dstp_hbm.at[pl.ds(e0, K)], didx)
            pltpu.sync_copy(buf, acc.at[didx], add=True)
            return carry
        lax.fori_loop(0, steps, step, 0)
        plsc.subcore_barrier()
        pltpu.sync_copy(acc.at[pl.ds(s * RO_STRIDE, RO_LEN)],
                        out_hbm.at[c, pl.ds(s * RO_STRIDE, RO_LEN)])

    return deg_kernel


@functools.lru_cache(maxsize=None)
def _make_agg(N, D, E_pad):
    EPT = E_pad // NW
    steps = EPT // K
    NR = ((N + 1 + NS * K - 1) // (NS * K)) * (NS * K)
    RZ = NR // NS
    RO_STRIDE = 8 * (N // (8 * NS))
    RO_LEN = N - (NS - 1) * RO_STRIDE
    mesh = plsc.VectorSubcoreMesh(core_axis_name="c", subcore_axis_name="s", num_cores=NC, num_subcores=NS)

    @functools.partial(
        pl.kernel,
        out_type=jax.ShapeDtypeStruct((NC, N, D), jnp.float32),
        mesh=mesh,
        scratch_types=[
            pltpu.VMEM((K,), jnp.int32),
            pltpu.VMEM((K,), jnp.int32),
            pltpu.VMEM((K, D), jnp.float32),
            pltpu.VMEM_SHARED((NR, D), jnp.float32),
            pltpu.SemaphoreType.DMA,
        ],
    )
    def agg_kernel(hs_hbm, srcp_hbm, dstp_hbm, out_hbm, sidx, didx, rows, acc, sem):
        c = lax.axis_index("c")
        s = lax.axis_index("s")
        tid = c * NS + s
        _fill_rows(rows, K, D, 0.0)
        for i in range(RZ // K):
            pltpu.sync_copy(rows, acc.at[pl.ds(s * RZ + i * K, K)])
        plsc.subcore_barrier()

        def step(i, carry):
            e0 = tid * EPT + i * K
            pltpu.sync_copy(srcp_hbm.at[pl.ds(e0, K)], sidx)
            pltpu.sync_copy(dstp_hbm.at[pl.ds(e0, K)], didx)
            pltpu.async_copy(hs_hbm.at[sidx], rows, sem).wait()
            pltpu.sync_copy(rows, acc.at[didx], add=True)
            return carry
        lax.fori_loop(0, steps, step, 0)
        plsc.subcore_barrier()
        pltpu.sync_copy(acc.at[pl.ds(s * RO_STRIDE, RO_LEN)],
                        out_hbm.at[c, pl.ds(s * RO_STRIDE, RO_LEN)])

    return agg_kernel


def _dinv_from(degs_ref):
    deg = degs_ref[0, :, 0] + degs_ref[1, :, 0] + 1.0
    return lax.rsqrt(deg)


def _t1_body(x_ref, w_ref, degs_ref, o_ref):
    dinv = _dinv_from(degs_ref)
    h = jnp.dot(x_ref[...], w_ref[...], preferred_element_type=jnp.float32)
    o_ref[...] = h * dinv[:, None]


def _t2_body(p_ref, hs_ref, degs_ref, b_ref, w_ref, o_ref):
    dinv = _dinv_from(degs_ref)
    t = (p_ref[0] + p_ref[1] + hs_ref[...]) * dinv[:, None] + b_ref[0]
    t = jnp.maximum(t, 0.0)
    r = jnp.dot(t, w_ref[...], preferred_element_type=jnp.float32) * dinv[:, None]
    d = r.shape[1]
    o_ref[...] = jnp.concatenate([r, jnp.zeros_like(r)], axis=1) if d * 2 == o_ref.shape[1] else r


def _t3_body(p_ref, hs_ref, degs_ref, b_ref, o_ref):
    d = o_ref.shape[1]
    dinv = _dinv_from(degs_ref)
    agg = (p_ref[0] + p_ref[1] + hs_ref[...])[:, :d]
    z = agg * dinv[:, None] + b_ref[0]
    m = jnp.max(z, axis=1, keepdims=True)
    e = jnp.exp(z - m)
    o_ref[...] = z - m - jnp.log(jnp.sum(e, axis=1, keepdims=True))


_BN = 1000


def _t1(x, W1, degs):
    N, D_in = x.shape
    D_h = W1.shape[1]
    grid = N // _BN
    return pl.pallas_call(
        _t1_body,
        grid=(grid,),
        in_specs=[
            pl.BlockSpec((_BN, D_in), lambda i: (i, 0)),
            pl.BlockSpec((D_in, D_h), lambda i: (0, 0)),
            pl.BlockSpec((NC, _BN, DEG_W), lambda i: (0, i, 0)),
        ],
        out_specs=pl.BlockSpec((_BN, D_h), lambda i: (i, 0)),
        out_shape=jax.ShapeDtypeStruct((N, D_h), jnp.float32),
    )(x, W1, degs)


def _t2(p, hs1, degs, b1, W2):
    N, D_h = hs1.shape
    D_out = W2.shape[1]
    grid = N // _BN
    return pl.pallas_call(
        _t2_body,
        grid=(grid,),
        in_specs=[
            pl.BlockSpec((NC, _BN, D_h), lambda i: (0, i, 0)),
            pl.BlockSpec((_BN, D_h), lambda i: (i, 0)),
            pl.BlockSpec((NC, _BN, DEG_W), lambda i: (0, i, 0)),
            pl.BlockSpec((1, D_h), lambda i: (0, 0)),
            pl.BlockSpec((D_h, D_out), lambda i: (0, 0)),
        ],
        out_specs=pl.BlockSpec((_BN, 2 * D_out), lambda i: (i, 0)),
        out_shape=jax.ShapeDtypeStruct((N, 2 * D_out), jnp.float32),
    )(p, hs1, degs, b1.reshape(1, D_h), W2)


def _t3(p, hs2, degs, b2):
    N = hs2.shape[0]
    D_out = b2.shape[0]
    grid = N // _BN
    return pl.pallas_call(
        _t3_body,
        grid=(grid,),
        in_specs=[
            pl.BlockSpec((NC, _BN, 2 * D_out), lambda i: (0, i, 0)),
            pl.BlockSpec((_BN, 2 * D_out), lambda i: (i, 0)),
            pl.BlockSpec((NC, _BN, DEG_W), lambda i: (0, i, 0)),
            pl.BlockSpec((1, D_out), lambda i: (0, 0)),
        ],
        out_specs=pl.BlockSpec((_BN, D_out), lambda i: (i, 0)),
        out_shape=jax.ShapeDtypeStruct((N, D_out), jnp.float32),
    )(p, hs2, degs, b2.reshape(1, D_out))


def kernel(x, edge_index, W1, b1, W2, b2):
    N = x.shape[0]
    E = edge_index.shape[1]
    chunk = NW * K
    E_pad = ((E + chunk - 1) // chunk) * chunk
    pad = E_pad - E
    src = edge_index[0]
    dst = edge_index[1]
    srcp = jnp.concatenate([src, jnp.zeros((pad,), jnp.int32)])
    dstp = jnp.concatenate([dst, jnp.full((pad,), N, jnp.int32)])

    degs = _make_deg(N, E_pad)(dstp)
    hs1 = _t1(x, W1, degs)
    p1 = _make_agg(N, hs1.shape[1], E_pad)(hs1, srcp, dstp)
    hs2 = _t2(p1, hs1, degs, b1, W2)
    p2 = _make_agg(N, hs2.shape[1], E_pad)(hs2, srcp, dstp)
    return _t3(p2, hs2, degs, b2)
